# baseline (speedup 1.0000x reference)
import functools

import jax
import jax.numpy as jnp
from jax import lax
from jax.experimental import pallas as pl
from jax.experimental.pallas import tpu as pltpu
from jax.experimental.pallas import tpu_sc as plsc

N_NODES = 10000
N_EDGES = 320000
D = 128

NC = 2
NS = 16
NW = NC * NS
EPW = N_EDGES // NW
K = 50
G = 5
NG = EPW // (G * K)
NCH = NG * G
NP = 10240
RPS = NP // NS
ZR = 80

K3 = 80
G3 = 5
NG3 = EPW // (G3 * K3)

_mesh = plsc.VectorSubcoreMesh(
    core_axis_name="c", subcore_axis_name="s", num_cores=NC, num_subcores=NS
)


def _sc_aggregate_body(x_hbm, src_hbm, dst_hbm, out_hbm,
                       sidx_v, didx_v, rows_v, zeros_v, acc):
    c = lax.axis_index("c")
    s = lax.axis_index("s")
    wid = s * NC + c

    z16 = jnp.zeros((16,), jnp.float32)

    @pl.loop(0, ZR)
    def _fill(i):
        for j in range(D // 16):
            zeros_v[i, pl.ds(j * 16, 16)] = z16

    for t in range(RPS // ZR):
        pltpu.sync_copy(zeros_v, acc.at[pl.ds(s * RPS + t * ZR, ZR)])
    plsc.subcore_barrier()

    @pl.loop(0, NG)
    def _group(g):
        pltpu.sync_copy(src_hbm.at[wid, g], sidx_v)
        pltpu.sync_copy(dst_hbm.at[wid, g], didx_v)
        for j in range(G):
            pltpu.sync_copy(x_hbm.at[sidx_v.at[j]], rows_v)
            pltpu.sync_copy(rows_v, acc.at[didx_v.at[j]], add=True)

    plsc.subcore_barrier()
    pltpu.sync_copy(acc.at[pl.ds(s * RPS, RPS)],
                    out_hbm.at[c, pl.ds(s * RPS, RPS)])


_sc_aggregate = pl.kernel(
    _sc_aggregate_body,
    out_type=jax.ShapeDtypeStruct((NC, NP, D), jnp.float32),
    mesh=_mesh,
    scratch_types=[
        pltpu.VMEM((G, K), jnp.int32),
        pltpu.VMEM((G, K), jnp.int32),
        pltpu.VMEM((K, D), jnp.float32),
        pltpu.VMEM((ZR, D), jnp.float32),
        pltpu.MemorySpace.VMEM_SHARED((NP, D), jnp.float32),
    ],
)


def _tc_mlp_body(scale_ref, x_ref, p0_ref, p1_ref, w1_ref, b1_ref,
                 w2_ref, b2_ref, wfc_ref, bs_ref, s_ref):
    h = x_ref[...] * scale_ref[0] + p0_ref[...] + p1_ref[...]
    h = jnp.maximum(
        jnp.dot(h, w1_ref[...], preferred_element_type=jnp.float32)
        + b1_ref[...], 0.0)
    h = jnp.maximum(
        jnp.dot(h, w2_ref[...], preferred_element_type=jnp.float32)
        + b2_ref[...], 0.0)
    s_ref[...] = (
        jnp.dot(h, wfc_ref[...], preferred_element_type=jnp.float32)
        + bs_ref[...])


_RB = 1000

_tc_mlp = pl.pallas_call(
    _tc_mlp_body,
    grid=(N_NODES // _RB,),
    in_specs=[
        pl.BlockSpec(memory_space=pltpu.MemorySpace.SMEM),
        pl.BlockSpec((_RB, D), lambda i: (i, 0)),
        pl.BlockSpec((_RB, D), lambda i: (i, 0)),
        pl.BlockSpec((_RB, D), lambda i: (i, 0)),
        pl.BlockSpec((D, D), lambda i: (0, 0)),
        pl.BlockSpec((1, D), lambda i: (0, 0)),
        pl.BlockSpec((D, D), lambda i: (0, 0)),
        pl.BlockSpec((1, D), lambda i: (0, 0)),
        pl.BlockSpec((D, 8), lambda i: (0, 0)),
        pl.BlockSpec((1, 8), lambda i: (0, 0)),
    ],
    out_specs=pl.BlockSpec((_RB, 8), lambda i: (i, 0)),
    out_shape=jax.ShapeDtypeStruct((N_NODES, 8), jnp.float32),
)


def _sc_scores_body(s1_hbm, s2_hbm, src_hbm, dst_hbm, out_hbm,
                    sidx_v, didx_v, out_v, s1_sh, s2_sh):
    c = lax.axis_index("c")
    s = lax.axis_index("s")
    wid = s * NC + c

    @pl.when(s == 0)
    def _stage():
        pltpu.sync_copy(s1_hbm, s1_sh)
        pltpu.sync_copy(s2_hbm, s2_sh)

    plsc.subcore_barrier()

    @pl.loop(0, NG3)
    def _group(g):
        pltpu.sync_copy(src_hbm.at[wid, g], sidx_v)
        pltpu.sync_copy(dst_hbm.at[wid, g], didx_v)
        for j in range(G3):
            chunk = out_v.at[pl.ds(g * G3 * K3 + j * K3, K3)]
            pltpu.sync_copy(s1_sh.at[sidx_v.at[j]], chunk)
            pltpu.sync_copy(s2_sh.at[didx_v.at[j]], chunk, add=True)

    pltpu.sync_copy(out_v, out_hbm.at[pl.ds(wid * EPW, EPW)])


_sc_scores = pl.kernel(
    _sc_scores_body,
    out_type=jax.ShapeDtypeStruct((N_EDGES,), jnp.float32),
    mesh=_mesh,
    scratch_types=[
        pltpu.VMEM((G3, K3), jnp.int32),
        pltpu.VMEM((G3, K3), jnp.int32),
        pltpu.VMEM((EPW,), jnp.float32),
        pltpu.MemorySpace.VMEM_SHARED((N_NODES,), jnp.float32),
        pltpu.MemorySpace.VMEM_SHARED((N_NODES,), jnp.float32),
    ],
)


def kernel(x, edge_index, eps, W1, b1, W2, b2, Wfc, bfc):
    src = edge_index[0].astype(jnp.int32)
    dst = edge_index[1].astype(jnp.int32)
    src3 = src.reshape(NW, NG, G, K)
    dst3 = dst.reshape(NW, NG, G, K)

    parts = _sc_aggregate(x, src3, dst3)

    scale = (1.0 + eps).reshape(1).astype(jnp.float32)
    wfc2 = jnp.pad(Wfc.reshape(2, D).T, ((0, 0), (0, 6)))
    bs = jnp.zeros((1, 8), jnp.float32).at[0, 0].set(bfc[0])
    S = _tc_mlp(scale, x, parts[0], parts[1], W1, b1.reshape(1, D),
                W2, b2.reshape(1, D), wfc2, bs)

    srcS = src.reshape(NW, NG3, G3, K3)
    dstS = dst.reshape(NW, NG3, G3, K3)
    return _sc_scores(S[:, 0], S[:, 1], srcS, dstS)


# speedup vs baseline: 1.1887x; 1.1887x over previous
import functools

import jax
import jax.numpy as jnp
from jax import lax
from jax.experimental import pallas as pl
from jax.experimental.pallas import tpu as pltpu
from jax.experimental.pallas import tpu_sc as plsc

N_NODES = 10000
N_EDGES = 320000
D = 128

NC = 2
NS = 16
NW = NC * NS
EPW = N_EDGES // NW
K = 80
G = 5
NG = EPW // (G * K)
NCH = NG * G
NP = 10240
RPS = NP // NS
ZR = 80

K3 = 80
G3 = 5
NG3 = EPW // (G3 * K3)

_mesh = plsc.VectorSubcoreMesh(
    core_axis_name="c", subcore_axis_name="s", num_cores=NC, num_subcores=NS
)


def _sc_aggregate_body(x_hbm, src_hbm, dst_hbm, out_hbm,
                       sidx_v, didx_v, rows_v, zeros_v, acc):
    c = lax.axis_index("c")
    s = lax.axis_index("s")
    wid = s * NC + c

    z16 = jnp.zeros((16,), jnp.float32)

    @pl.loop(0, ZR)
    def _fill(i):
        for j in range(D // 16):
            zeros_v[i, pl.ds(j * 16, 16)] = z16

    for t in range(RPS // ZR):
        pltpu.sync_copy(zeros_v, acc.at[pl.ds(s * RPS + t * ZR, ZR)])
    plsc.subcore_barrier()

    @pl.loop(0, NG)
    def _group(g):
        pltpu.sync_copy(src_hbm.at[wid, g], sidx_v)
        pltpu.sync_copy(dst_hbm.at[wid, g], didx_v)
        for j in range(G):
            pltpu.sync_copy(x_hbm.at[sidx_v.at[j]], rows_v)
            pltpu.sync_copy(rows_v, acc.at[didx_v.at[j]], add=True)

    plsc.subcore_barrier()
    pltpu.sync_copy(acc.at[pl.ds(s * RPS, RPS)],
                    out_hbm.at[c, pl.ds(s * RPS, RPS)])


_sc_aggregate = pl.kernel(
    _sc_aggregate_body,
    out_type=jax.ShapeDtypeStruct((NC, NP, D), jnp.float32),
    mesh=_mesh,
    scratch_types=[
        pltpu.VMEM((G, K), jnp.int32),
        pltpu.VMEM((G, K), jnp.int32),
        pltpu.VMEM((K, D), jnp.float32),
        pltpu.VMEM((ZR, D), jnp.float32),
        pltpu.MemorySpace.VMEM_SHARED((NP, D), jnp.float32),
    ],
)


def _tc_mlp_body(scale_ref, x_ref, p0_ref, p1_ref, w1_ref, b1_ref,
                 w2_ref, b2_ref, wfc_ref, bs_ref, s_ref):
    h = x_ref[...] * scale_ref[0] + p0_ref[...] + p1_ref[...]
    h = jnp.maximum(
        jnp.dot(h, w1_ref[...], preferred_element_type=jnp.float32)
        + b1_ref[...], 0.0)
    h = jnp.maximum(
        jnp.dot(h, w2_ref[...], preferred_element_type=jnp.float32)
        + b2_ref[...], 0.0)
    s_ref[...] = (
        jnp.dot(h, wfc_ref[...], preferred_element_type=jnp.float32)
        + bs_ref[...])


_RB = 1000

_tc_mlp = pl.pallas_call(
    _tc_mlp_body,
    grid=(N_NODES // _RB,),
    in_specs=[
        pl.BlockSpec(memory_space=pltpu.MemorySpace.SMEM),
        pl.BlockSpec((_RB, D), lambda i: (i, 0)),
        pl.BlockSpec((_RB, D), lambda i: (i, 0)),
        pl.BlockSpec((_RB, D), lambda i: (i, 0)),
        pl.BlockSpec((D, D), lambda i: (0, 0)),
        pl.BlockSpec((1, D), lambda i: (0, 0)),
        pl.BlockSpec((D, D), lambda i: (0, 0)),
        pl.BlockSpec((1, D), lambda i: (0, 0)),
        pl.BlockSpec((D, 8), lambda i: (0, 0)),
        pl.BlockSpec((1, 8), lambda i: (0, 0)),
    ],
    out_specs=pl.BlockSpec((_RB, 8), lambda i: (i, 0)),
    out_shape=jax.ShapeDtypeStruct((N_NODES, 8), jnp.float32),
)


def _sc_scores_body(s1_hbm, s2_hbm, src_hbm, dst_hbm, out_hbm,
                    sidx_v, didx_v, out_v, s1_sh, s2_sh):
    c = lax.axis_index("c")
    s = lax.axis_index("s")
    wid = s * NC + c

    @pl.when(s == 0)
    def _stage():
        pltpu.sync_copy(s1_hbm, s1_sh)
        pltpu.sync_copy(s2_hbm, s2_sh)

    plsc.subcore_barrier()

    @pl.loop(0, NG3)
    def _group(g):
        pltpu.sync_copy(src_hbm.at[wid, g], sidx_v)
        pltpu.sync_copy(dst_hbm.at[wid, g], didx_v)
        for j in range(G3):
            chunk = out_v.at[pl.ds(g * G3 * K3 + j * K3, K3)]
            pltpu.sync_copy(s1_sh.at[sidx_v.at[j]], chunk)
            pltpu.sync_copy(s2_sh.at[didx_v.at[j]], chunk, add=True)

    pltpu.sync_copy(out_v, out_hbm.at[pl.ds(wid * EPW, EPW)])


_sc_scores = pl.kernel(
    _sc_scores_body,
    out_type=jax.ShapeDtypeStruct((N_EDGES,), jnp.float32),
    mesh=_mesh,
    scratch_types=[
        pltpu.VMEM((G3, K3), jnp.int32),
        pltpu.VMEM((G3, K3), jnp.int32),
        pltpu.VMEM((EPW,), jnp.float32),
        pltpu.MemorySpace.VMEM_SHARED((N_NODES,), jnp.float32),
        pltpu.MemorySpace.VMEM_SHARED((N_NODES,), jnp.float32),
    ],
)


def kernel(x, edge_index, eps, W1, b1, W2, b2, Wfc, bfc):
    src = edge_index[0].astype(jnp.int32)
    dst = edge_index[1].astype(jnp.int32)
    src3 = src.reshape(NW, NG, G, K)
    dst3 = dst.reshape(NW, NG, G, K)

    parts = _sc_aggregate(x, src3, dst3)

    scale = (1.0 + eps).reshape(1).astype(jnp.float32)
    wfc2 = jnp.pad(Wfc.reshape(2, D).T, ((0, 0), (0, 6)))
    bs = jnp.zeros((1, 8), jnp.float32).at[0, 0].set(bfc[0])
    S = _tc_mlp(scale, x, parts[0], parts[1], W1, b1.reshape(1, D),
                W2, b2.reshape(1, D), wfc2, bs)

    srcS = src.reshape(NW, NG3, G3, K3)
    dstS = dst.reshape(NW, NG3, G3, K3)
    return _sc_scores(S[:, 0], S[:, 1], srcS, dstS)


# speedup vs baseline: 1.6856x; 1.4180x over previous
import functools

import jax
import jax.numpy as jnp
from jax import lax
from jax.experimental import pallas as pl
from jax.experimental.pallas import tpu as pltpu
from jax.experimental.pallas import tpu_sc as plsc

N_NODES = 10000
N_EDGES = 320000
D = 128

NC = 2
NS = 16
NW = NC * NS
EPW = N_EDGES // NW
K = 80
G = 25
NG = EPW // (G * K)
NCH = NG * G
NP = 10240
RPS = NP // NS
ZR = 80

K3 = 80
G3 = 5
NG3 = EPW // (G3 * K3)

_mesh = plsc.VectorSubcoreMesh(
    core_axis_name="c", subcore_axis_name="s", num_cores=NC, num_subcores=NS
)


def _sc_aggregate_body(x_hbm, src_hbm, dst_hbm, out_hbm,
                       sidx_v, didx_v, rows0, rows1, zeros_v, acc,
                       sem0, sem1):
    c = lax.axis_index("c")
    s = lax.axis_index("s")
    wid = s * NC + c

    z16 = jnp.zeros((16,), jnp.float32)

    @pl.loop(0, ZR)
    def _fill(i):
        for j in range(D // 16):
            zeros_v[i, pl.ds(j * 16, 16)] = z16

    for t in range(RPS // ZR):
        pltpu.sync_copy(zeros_v, acc.at[pl.ds(s * RPS + t * ZR, ZR)])
    plsc.subcore_barrier()

    @pl.loop(0, NG)
    def _group(g):
        pltpu.sync_copy(src_hbm.at[wid, g], sidx_v)
        pltpu.sync_copy(dst_hbm.at[wid, g], didx_v)
        pltpu.async_copy(x_hbm.at[sidx_v.at[0]], rows0, sem0)
        pltpu.async_copy(x_hbm.at[sidx_v.at[1]], rows1, sem1)

        @pl.loop(0, (G - 3) // 2)
        def _pair(i):
            j0 = 2 * i
            pltpu.make_async_copy(x_hbm.at[sidx_v.at[j0]], rows0, sem0).wait()
            pltpu.sync_copy(rows0, acc.at[didx_v.at[j0]], add=True)
            pltpu.async_copy(x_hbm.at[sidx_v.at[j0 + 2]], rows0, sem0)
            pltpu.make_async_copy(x_hbm.at[sidx_v.at[j0 + 1]], rows1,
                                  sem1).wait()
            pltpu.sync_copy(rows1, acc.at[didx_v.at[j0 + 1]], add=True)
            pltpu.async_copy(x_hbm.at[sidx_v.at[j0 + 3]], rows1, sem1)

        pltpu.make_async_copy(x_hbm.at[sidx_v.at[G - 3]], rows0, sem0).wait()
        pltpu.sync_copy(rows0, acc.at[didx_v.at[G - 3]], add=True)
        pltpu.async_copy(x_hbm.at[sidx_v.at[G - 1]], rows0, sem0)
        pltpu.make_async_copy(x_hbm.at[sidx_v.at[G - 2]], rows1, sem1).wait()
        pltpu.sync_copy(rows1, acc.at[didx_v.at[G - 2]], add=True)
        pltpu.make_async_copy(x_hbm.at[sidx_v.at[G - 1]], rows0, sem0).wait()
        pltpu.sync_copy(rows0, acc.at[didx_v.at[G - 1]], add=True)

    plsc.subcore_barrier()
    pltpu.sync_copy(acc.at[pl.ds(s * RPS, RPS)],
                    out_hbm.at[c, pl.ds(s * RPS, RPS)])


_sc_aggregate = pl.kernel(
    _sc_aggregate_body,
    out_type=jax.ShapeDtypeStruct((NC, NP, D), jnp.float32),
    mesh=_mesh,
    scratch_types=[
        pltpu.VMEM((G, K), jnp.int32),
        pltpu.VMEM((G, K), jnp.int32),
        pltpu.VMEM((K, D), jnp.float32),
        pltpu.VMEM((K, D), jnp.float32),
        pltpu.VMEM((ZR, D), jnp.float32),
        pltpu.MemorySpace.VMEM_SHARED((NP, D), jnp.float32),
        pltpu.SemaphoreType.DMA,
        pltpu.SemaphoreType.DMA,
    ],
)


def _tc_mlp_body(scale_ref, x_ref, p0_ref, p1_ref, w1_ref, b1_ref,
                 w2_ref, b2_ref, wfc_ref, bs_ref, s_ref):
    h = x_ref[...] * scale_ref[0] + p0_ref[...] + p1_ref[...]
    h = jnp.maximum(
        jnp.dot(h, w1_ref[...], preferred_element_type=jnp.float32)
        + b1_ref[...], 0.0)
    h = jnp.maximum(
        jnp.dot(h, w2_ref[...], preferred_element_type=jnp.float32)
        + b2_ref[...], 0.0)
    s_ref[...] = (
        jnp.dot(h, wfc_ref[...], preferred_element_type=jnp.float32)
        + bs_ref[...])


_RB = 1000

_tc_mlp = pl.pallas_call(
    _tc_mlp_body,
    grid=(N_NODES // _RB,),
    in_specs=[
        pl.BlockSpec(memory_space=pltpu.MemorySpace.SMEM),
        pl.BlockSpec((_RB, D), lambda i: (i, 0)),
        pl.BlockSpec((_RB, D), lambda i: (i, 0)),
        pl.BlockSpec((_RB, D), lambda i: (i, 0)),
        pl.BlockSpec((D, D), lambda i: (0, 0)),
        pl.BlockSpec((1, D), lambda i: (0, 0)),
        pl.BlockSpec((D, D), lambda i: (0, 0)),
        pl.BlockSpec((1, D), lambda i: (0, 0)),
        pl.BlockSpec((D, 8), lambda i: (0, 0)),
        pl.BlockSpec((1, 8), lambda i: (0, 0)),
    ],
    out_specs=pl.BlockSpec((_RB, 8), lambda i: (i, 0)),
    out_shape=jax.ShapeDtypeStruct((N_NODES, 8), jnp.float32),
)


def _sc_scores_body(s1_hbm, s2_hbm, src_hbm, dst_hbm, out_hbm,
                    sidx_v, didx_v, out_v, s1_sh, s2_sh):
    c = lax.axis_index("c")
    s = lax.axis_index("s")
    wid = s * NC + c

    @pl.when(s == 0)
    def _stage():
        pltpu.sync_copy(s1_hbm, s1_sh)
        pltpu.sync_copy(s2_hbm, s2_sh)

    plsc.subcore_barrier()

    @pl.loop(0, NG3)
    def _group(g):
        pltpu.sync_copy(src_hbm.at[wid, g], sidx_v)
        pltpu.sync_copy(dst_hbm.at[wid, g], didx_v)
        for j in range(G3):
            chunk = out_v.at[pl.ds(g * G3 * K3 + j * K3, K3)]
            pltpu.sync_copy(s1_sh.at[sidx_v.at[j]], chunk)
            pltpu.sync_copy(s2_sh.at[didx_v.at[j]], chunk, add=True)

    pltpu.sync_copy(out_v, out_hbm.at[pl.ds(wid * EPW, EPW)])


_sc_scores = pl.kernel(
    _sc_scores_body,
    out_type=jax.ShapeDtypeStruct((N_EDGES,), jnp.float32),
    mesh=_mesh,
    scratch_types=[
        pltpu.VMEM((G3, K3), jnp.int32),
        pltpu.VMEM((G3, K3), jnp.int32),
        pltpu.VMEM((EPW,), jnp.float32),
        pltpu.MemorySpace.VMEM_SHARED((N_NODES,), jnp.float32),
        pltpu.MemorySpace.VMEM_SHARED((N_NODES,), jnp.float32),
    ],
)


def kernel(x, edge_index, eps, W1, b1, W2, b2, Wfc, bfc):
    src = edge_index[0].astype(jnp.int32)
    dst = edge_index[1].astype(jnp.int32)
    src3 = src.reshape(NW, NG, G, K)
    dst3 = dst.reshape(NW, NG, G, K)

    parts = _sc_aggregate(x, src3, dst3)

    scale = (1.0 + eps).reshape(1).astype(jnp.float32)
    wfc2 = jnp.pad(Wfc.reshape(2, D).T, ((0, 0), (0, 6)))
    bs = jnp.zeros((1, 8), jnp.float32).at[0, 0].set(bfc[0])
    S = _tc_mlp(scale, x, parts[0], parts[1], W1, b1.reshape(1, D),
                W2, b2.reshape(1, D), wfc2, bs)

    srcS = src.reshape(NW, NG3, G3, K3)
    dstS = dst.reshape(NW, NG3, G3, K3)
    return _sc_scores(S[:, 0], S[:, 1], srcS, dstS)


# speedup vs baseline: 1.9349x; 1.1479x over previous
import functools

import jax
import jax.numpy as jnp
from jax import lax
from jax.experimental import pallas as pl
from jax.experimental.pallas import tpu as pltpu
from jax.experimental.pallas import tpu_sc as plsc

N_NODES = 10000
N_EDGES = 320000
D = 128

NC = 2
NS = 16
NW = NC * NS
EPW = N_EDGES // NW
K = 80
G = 25
NG = EPW // (G * K)
NCH = NG * G
NP = 10240
RPS = NP // NS
ZR = 80

K3 = 80
G3 = 25
NG3 = EPW // (G3 * K3)

_mesh = plsc.VectorSubcoreMesh(
    core_axis_name="c", subcore_axis_name="s", num_cores=NC, num_subcores=NS
)


def _sc_aggregate_body(x_hbm, src_hbm, dst_hbm, out_hbm,
                       sidx_v, didx_v, rows0, rows1, zeros_v, acc,
                       sem0, sem1):
    c = lax.axis_index("c")
    s = lax.axis_index("s")
    wid = s * NC + c

    z16 = jnp.zeros((16,), jnp.float32)

    @pl.loop(0, ZR)
    def _fill(i):
        for j in range(D // 16):
            zeros_v[i, pl.ds(j * 16, 16)] = z16

    for t in range(RPS // ZR):
        pltpu.sync_copy(zeros_v, acc.at[pl.ds(s * RPS + t * ZR, ZR)])
    plsc.subcore_barrier()

    @pl.loop(0, NG)
    def _group(g):
        pltpu.sync_copy(src_hbm.at[wid, g], sidx_v)
        pltpu.sync_copy(dst_hbm.at[wid, g], didx_v)
        pltpu.async_copy(x_hbm.at[sidx_v.at[0]], rows0, sem0)
        pltpu.async_copy(x_hbm.at[sidx_v.at[1]], rows1, sem1)

        @pl.loop(0, (G - 3) // 2)
        def _pair(i):
            j0 = 2 * i
            pltpu.make_async_copy(x_hbm.at[sidx_v.at[j0]], rows0, sem0).wait()
            pltpu.sync_copy(rows0, acc.at[didx_v.at[j0]], add=True)
            pltpu.async_copy(x_hbm.at[sidx_v.at[j0 + 2]], rows0, sem0)
            pltpu.make_async_copy(x_hbm.at[sidx_v.at[j0 + 1]], rows1,
                                  sem1).wait()
            pltpu.sync_copy(rows1, acc.at[didx_v.at[j0 + 1]], add=True)
            pltpu.async_copy(x_hbm.at[sidx_v.at[j0 + 3]], rows1, sem1)

        pltpu.make_async_copy(x_hbm.at[sidx_v.at[G - 3]], rows0, sem0).wait()
        pltpu.sync_copy(rows0, acc.at[didx_v.at[G - 3]], add=True)
        pltpu.async_copy(x_hbm.at[sidx_v.at[G - 1]], rows0, sem0)
        pltpu.make_async_copy(x_hbm.at[sidx_v.at[G - 2]], rows1, sem1).wait()
        pltpu.sync_copy(rows1, acc.at[didx_v.at[G - 2]], add=True)
        pltpu.make_async_copy(x_hbm.at[sidx_v.at[G - 1]], rows0, sem0).wait()
        pltpu.sync_copy(rows0, acc.at[didx_v.at[G - 1]], add=True)

    plsc.subcore_barrier()
    pltpu.sync_copy(acc.at[pl.ds(s * RPS, RPS)],
                    out_hbm.at[c, pl.ds(s * RPS, RPS)])


_sc_aggregate = pl.kernel(
    _sc_aggregate_body,
    out_type=jax.ShapeDtypeStruct((NC, NP, D), jnp.float32),
    mesh=_mesh,
    scratch_types=[
        pltpu.VMEM((G, K), jnp.int32),
        pltpu.VMEM((G, K), jnp.int32),
        pltpu.VMEM((K, D), jnp.float32),
        pltpu.VMEM((K, D), jnp.float32),
        pltpu.VMEM((ZR, D), jnp.float32),
        pltpu.MemorySpace.VMEM_SHARED((NP, D), jnp.float32),
        pltpu.SemaphoreType.DMA,
        pltpu.SemaphoreType.DMA,
    ],
)


def _tc_mlp_body(scale_ref, x_ref, p0_ref, p1_ref, w1_ref, b1_ref,
                 w2_ref, b2_ref, wfc_ref, bs_ref, s_ref):
    h = x_ref[...] * scale_ref[0] + p0_ref[...] + p1_ref[...]
    h = jnp.maximum(
        jnp.dot(h, w1_ref[...], preferred_element_type=jnp.float32)
        + b1_ref[...], 0.0)
    h = jnp.maximum(
        jnp.dot(h, w2_ref[...], preferred_element_type=jnp.float32)
        + b2_ref[...], 0.0)
    s_ref[...] = (
        jnp.dot(h, wfc_ref[...], preferred_element_type=jnp.float32)
        + bs_ref[...])


_RB = 1000

_tc_mlp = pl.pallas_call(
    _tc_mlp_body,
    grid=(N_NODES // _RB,),
    in_specs=[
        pl.BlockSpec(memory_space=pltpu.MemorySpace.SMEM),
        pl.BlockSpec((_RB, D), lambda i: (i, 0)),
        pl.BlockSpec((_RB, D), lambda i: (i, 0)),
        pl.BlockSpec((_RB, D), lambda i: (i, 0)),
        pl.BlockSpec((D, D), lambda i: (0, 0)),
        pl.BlockSpec((1, D), lambda i: (0, 0)),
        pl.BlockSpec((D, D), lambda i: (0, 0)),
        pl.BlockSpec((1, D), lambda i: (0, 0)),
        pl.BlockSpec((D, 8), lambda i: (0, 0)),
        pl.BlockSpec((1, 8), lambda i: (0, 0)),
    ],
    out_specs=pl.BlockSpec((_RB, 8), lambda i: (i, 0)),
    out_shape=jax.ShapeDtypeStruct((N_NODES, 8), jnp.float32),
)


def _sc_scores_body(s1_hbm, s2_hbm, src_hbm, dst_hbm, out_hbm,
                    sidx_v, didx_v, out_v, s1_sh, s2_sh, sem0, sem1):
    c = lax.axis_index("c")
    s = lax.axis_index("s")
    wid = s * NC + c

    @pl.when(s == 0)
    def _stage():
        pltpu.sync_copy(s1_hbm, s1_sh)
        pltpu.sync_copy(s2_hbm, s2_sh)

    plsc.subcore_barrier()

    @pl.loop(0, NG3)
    def _group(g):
        pltpu.sync_copy(src_hbm.at[wid, g], sidx_v)
        pltpu.sync_copy(dst_hbm.at[wid, g], didx_v)
        base = g * G3 * K3

        def _sl(j):
            return out_v.at[pl.ds(base + j * K3, K3)]

        pltpu.async_copy(s1_sh.at[sidx_v.at[0]], _sl(0), sem0)
        pltpu.async_copy(s1_sh.at[sidx_v.at[1]], _sl(1), sem1)

        @pl.loop(0, (G3 - 3) // 2)
        def _pair(i):
            j0 = 2 * i
            pltpu.make_async_copy(s1_sh.at[sidx_v.at[j0]], _sl(j0),
                                  sem0).wait()
            pltpu.async_copy(s1_sh.at[sidx_v.at[j0 + 2]], _sl(j0 + 2), sem0)
            pltpu.sync_copy(s2_sh.at[didx_v.at[j0]], _sl(j0), add=True)
            pltpu.make_async_copy(s1_sh.at[sidx_v.at[j0 + 1]], _sl(j0 + 1),
                                  sem1).wait()
            pltpu.async_copy(s1_sh.at[sidx_v.at[j0 + 3]], _sl(j0 + 3), sem1)
            pltpu.sync_copy(s2_sh.at[didx_v.at[j0 + 1]], _sl(j0 + 1),
                            add=True)

        pltpu.make_async_copy(s1_sh.at[sidx_v.at[G3 - 3]], _sl(G3 - 3),
                              sem0).wait()
        pltpu.async_copy(s1_sh.at[sidx_v.at[G3 - 1]], _sl(G3 - 1), sem0)
        pltpu.sync_copy(s2_sh.at[didx_v.at[G3 - 3]], _sl(G3 - 3), add=True)
        pltpu.make_async_copy(s1_sh.at[sidx_v.at[G3 - 2]], _sl(G3 - 2),
                              sem1).wait()
        pltpu.sync_copy(s2_sh.at[didx_v.at[G3 - 2]], _sl(G3 - 2), add=True)
        pltpu.make_async_copy(s1_sh.at[sidx_v.at[G3 - 1]], _sl(G3 - 1),
                              sem0).wait()
        pltpu.sync_copy(s2_sh.at[didx_v.at[G3 - 1]], _sl(G3 - 1), add=True)

    pltpu.sync_copy(out_v, out_hbm.at[pl.ds(wid * EPW, EPW)])


_sc_scores = pl.kernel(
    _sc_scores_body,
    out_type=jax.ShapeDtypeStruct((N_EDGES,), jnp.float32),
    mesh=_mesh,
    scratch_types=[
        pltpu.VMEM((G3, K3), jnp.int32),
        pltpu.VMEM((G3, K3), jnp.int32),
        pltpu.VMEM((EPW,), jnp.float32),
        pltpu.MemorySpace.VMEM_SHARED((N_NODES,), jnp.float32),
        pltpu.MemorySpace.VMEM_SHARED((N_NODES,), jnp.float32),
        pltpu.SemaphoreType.DMA,
        pltpu.SemaphoreType.DMA,
    ],
)


def kernel(x, edge_index, eps, W1, b1, W2, b2, Wfc, bfc):
    src = edge_index[0].astype(jnp.int32)
    dst = edge_index[1].astype(jnp.int32)
    src3 = src.reshape(NW, NG, G, K)
    dst3 = dst.reshape(NW, NG, G, K)

    parts = _sc_aggregate(x, src3, dst3)

    scale = (1.0 + eps).reshape(1).astype(jnp.float32)
    wfc2 = jnp.pad(Wfc.reshape(2, D).T, ((0, 0), (0, 6)))
    bs = jnp.zeros((1, 8), jnp.float32).at[0, 0].set(bfc[0])
    S = _tc_mlp(scale, x, parts[0], parts[1], W1, b1.reshape(1, D),
                W2, b2.reshape(1, D), wfc2, bs)

    srcS = src.reshape(NW, NG3, G3, K3)
    dstS = dst.reshape(NW, NG3, G3, K3)
    return _sc_scores(S[:, 0], S[:, 1], srcS, dstS)


# speedup vs baseline: 2.1323x; 1.1020x over previous
import functools

import jax
import jax.numpy as jnp
from jax import lax
from jax.experimental import pallas as pl
from jax.experimental.pallas import tpu as pltpu
from jax.experimental.pallas import tpu_sc as plsc

N_NODES = 10000
N_EDGES = 320000
D = 128

NC = 2
NS = 16
NW = NC * NS
EPW = N_EDGES // NW
K = 80
G = 25
NG = EPW // (G * K)
NCH = NG * G
NP = 10240
RPS = NP // NS
ZR = 80

K3 = 80
G3 = 25
NG3 = EPW // (G3 * K3)

_mesh = plsc.VectorSubcoreMesh(
    core_axis_name="c", subcore_axis_name="s", num_cores=NC, num_subcores=NS
)


NBUF = 4


def _sc_aggregate_body(x_hbm, src_hbm, dst_hbm, out_hbm,
                       sidx_v, didx_v, rows0, rows1, rows2, rows3,
                       acc, sem0, sem1, sem2, sem3):
    c = lax.axis_index("c")
    s = lax.axis_index("s")
    wid = s * NC + c
    rows = (rows0, rows1, rows2, rows3)
    sems = (sem0, sem1, sem2, sem3)

    z16 = jnp.zeros((16,), jnp.float32)

    @pl.loop(0, ZR)
    def _fill(i):
        for j in range(D // 16):
            rows0[i, pl.ds(j * 16, 16)] = z16

    for t in range(RPS // ZR):
        pltpu.sync_copy(rows0, acc.at[pl.ds(s * RPS + t * ZR, ZR)])
    plsc.subcore_barrier()

    @pl.loop(0, NG)
    def _group(g):
        pltpu.sync_copy(src_hbm.at[wid, g], sidx_v)
        pltpu.sync_copy(dst_hbm.at[wid, g], didx_v)
        for b in range(NBUF):
            pltpu.async_copy(x_hbm.at[sidx_v.at[b]], rows[b], sems[b])

        @pl.loop(0, (G - NBUF - 1) // NBUF)
        def _quad(i):
            j0 = NBUF * i
            for b in range(NBUF):
                pltpu.make_async_copy(x_hbm.at[sidx_v.at[j0 + b]], rows[b],
                                      sems[b]).wait()
                pltpu.sync_copy(rows[b], acc.at[didx_v.at[j0 + b]], add=True)
                pltpu.async_copy(x_hbm.at[sidx_v.at[j0 + NBUF + b]], rows[b],
                                 sems[b])

        pltpu.make_async_copy(x_hbm.at[sidx_v.at[G - 5]], rows[0],
                              sems[0]).wait()
        pltpu.sync_copy(rows[0], acc.at[didx_v.at[G - 5]], add=True)
        pltpu.async_copy(x_hbm.at[sidx_v.at[G - 1]], rows[0], sems[0])
        for b in range(1, NBUF):
            pltpu.make_async_copy(x_hbm.at[sidx_v.at[G - 5 + b]], rows[b],
                                  sems[b]).wait()
            pltpu.sync_copy(rows[b], acc.at[didx_v.at[G - 5 + b]], add=True)
        pltpu.make_async_copy(x_hbm.at[sidx_v.at[G - 1]], rows[0],
                              sems[0]).wait()
        pltpu.sync_copy(rows[0], acc.at[didx_v.at[G - 1]], add=True)

    plsc.subcore_barrier()
    pltpu.sync_copy(acc.at[pl.ds(s * RPS, RPS)],
                    out_hbm.at[c, pl.ds(s * RPS, RPS)])


_sc_aggregate = pl.kernel(
    _sc_aggregate_body,
    out_type=jax.ShapeDtypeStruct((NC, NP, D), jnp.float32),
    mesh=_mesh,
    scratch_types=[
        pltpu.VMEM((G, K), jnp.int32),
        pltpu.VMEM((G, K), jnp.int32),
        pltpu.VMEM((K, D), jnp.float32),
        pltpu.VMEM((K, D), jnp.float32),
        pltpu.VMEM((K, D), jnp.float32),
        pltpu.VMEM((K, D), jnp.float32),
        pltpu.MemorySpace.VMEM_SHARED((NP, D), jnp.float32),
        pltpu.SemaphoreType.DMA,
        pltpu.SemaphoreType.DMA,
        pltpu.SemaphoreType.DMA,
        pltpu.SemaphoreType.DMA,
    ],
)


def _tc_mlp_body(scale_ref, x_ref, p0_ref, p1_ref, w1_ref, b1_ref,
                 w2_ref, b2_ref, wfc_ref, bs_ref, s_ref):
    h = x_ref[...] * scale_ref[0] + p0_ref[...] + p1_ref[...]
    h = jnp.maximum(
        jnp.dot(h, w1_ref[...], preferred_element_type=jnp.float32)
        + b1_ref[...], 0.0)
    h = jnp.maximum(
        jnp.dot(h, w2_ref[...], preferred_element_type=jnp.float32)
        + b2_ref[...], 0.0)
    s_ref[...] = (
        jnp.dot(h, wfc_ref[...], preferred_element_type=jnp.float32)
        + bs_ref[...])


_RB = 1000

_tc_mlp = pl.pallas_call(
    _tc_mlp_body,
    grid=(N_NODES // _RB,),
    in_specs=[
        pl.BlockSpec(memory_space=pltpu.MemorySpace.SMEM),
        pl.BlockSpec((_RB, D), lambda i: (i, 0)),
        pl.BlockSpec((_RB, D), lambda i: (i, 0)),
        pl.BlockSpec((_RB, D), lambda i: (i, 0)),
        pl.BlockSpec((D, D), lambda i: (0, 0)),
        pl.BlockSpec((1, D), lambda i: (0, 0)),
        pl.BlockSpec((D, D), lambda i: (0, 0)),
        pl.BlockSpec((1, D), lambda i: (0, 0)),
        pl.BlockSpec((D, 8), lambda i: (0, 0)),
        pl.BlockSpec((1, 8), lambda i: (0, 0)),
    ],
    out_specs=pl.BlockSpec((_RB, 8), lambda i: (i, 0)),
    out_shape=jax.ShapeDtypeStruct((N_NODES, 8), jnp.float32),
)


def _sc_scores_body(s1_hbm, s2_hbm, src_hbm, dst_hbm, out_hbm,
                    sidx_v, didx_v, out_v, s1_sh, s2_sh, sem0, sem1):
    c = lax.axis_index("c")
    s = lax.axis_index("s")
    wid = s * NC + c

    @pl.when(s == 0)
    def _stage():
        pltpu.sync_copy(s1_hbm, s1_sh)
        pltpu.sync_copy(s2_hbm, s2_sh)

    plsc.subcore_barrier()

    @pl.loop(0, NG3)
    def _group(g):
        pltpu.sync_copy(src_hbm.at[wid, g], sidx_v)
        pltpu.sync_copy(dst_hbm.at[wid, g], didx_v)
        base = g * G3 * K3

        def _sl(j):
            return out_v.at[pl.ds(base + j * K3, K3)]

        pltpu.async_copy(s1_sh.at[sidx_v.at[0]], _sl(0), sem0)
        pltpu.async_copy(s1_sh.at[sidx_v.at[1]], _sl(1), sem1)

        @pl.loop(0, (G3 - 3) // 2)
        def _pair(i):
            j0 = 2 * i
            pltpu.make_async_copy(s1_sh.at[sidx_v.at[j0]], _sl(j0),
                                  sem0).wait()
            pltpu.async_copy(s1_sh.at[sidx_v.at[j0 + 2]], _sl(j0 + 2), sem0)
            pltpu.sync_copy(s2_sh.at[didx_v.at[j0]], _sl(j0), add=True)
            pltpu.make_async_copy(s1_sh.at[sidx_v.at[j0 + 1]], _sl(j0 + 1),
                                  sem1).wait()
            pltpu.async_copy(s1_sh.at[sidx_v.at[j0 + 3]], _sl(j0 + 3), sem1)
            pltpu.sync_copy(s2_sh.at[didx_v.at[j0 + 1]], _sl(j0 + 1),
                            add=True)

        pltpu.make_async_copy(s1_sh.at[sidx_v.at[G3 - 3]], _sl(G3 - 3),
                              sem0).wait()
        pltpu.async_copy(s1_sh.at[sidx_v.at[G3 - 1]], _sl(G3 - 1), sem0)
        pltpu.sync_copy(s2_sh.at[didx_v.at[G3 - 3]], _sl(G3 - 3), add=True)
        pltpu.make_async_copy(s1_sh.at[sidx_v.at[G3 - 2]], _sl(G3 - 2),
                              sem1).wait()
        pltpu.sync_copy(s2_sh.at[didx_v.at[G3 - 2]], _sl(G3 - 2), add=True)
        pltpu.make_async_copy(s1_sh.at[sidx_v.at[G3 - 1]], _sl(G3 - 1),
                              sem0).wait()
        pltpu.sync_copy(s2_sh.at[didx_v.at[G3 - 1]], _sl(G3 - 1), add=True)

    pltpu.sync_copy(out_v, out_hbm.at[pl.ds(wid * EPW, EPW)])


_sc_scores = pl.kernel(
    _sc_scores_body,
    out_type=jax.ShapeDtypeStruct((N_EDGES,), jnp.float32),
    mesh=_mesh,
    scratch_types=[
        pltpu.VMEM((G3, K3), jnp.int32),
        pltpu.VMEM((G3, K3), jnp.int32),
        pltpu.VMEM((EPW,), jnp.float32),
        pltpu.MemorySpace.VMEM_SHARED((N_NODES,), jnp.float32),
        pltpu.MemorySpace.VMEM_SHARED((N_NODES,), jnp.float32),
        pltpu.SemaphoreType.DMA,
        pltpu.SemaphoreType.DMA,
    ],
)


def kernel(x, edge_index, eps, W1, b1, W2, b2, Wfc, bfc):
    src = edge_index[0].astype(jnp.int32)
    dst = edge_index[1].astype(jnp.int32)
    src3 = src.reshape(NW, NG, G, K)
    dst3 = dst.reshape(NW, NG, G, K)

    parts = _sc_aggregate(x, src3, dst3)

    scale = (1.0 + eps).reshape(1).astype(jnp.float32)
    wfc2 = jnp.pad(Wfc.reshape(2, D).T, ((0, 0), (0, 6)))
    bs = jnp.zeros((1, 8), jnp.float32).at[0, 0].set(bfc[0])
    S = _tc_mlp(scale, x, parts[0], parts[1], W1, b1.reshape(1, D),
                W2, b2.reshape(1, D), wfc2, bs)

    srcS = src.reshape(NW, NG3, G3, K3)
    dstS = dst.reshape(NW, NG3, G3, K3)
    return _sc_scores(S[:, 0], S[:, 1], srcS, dstS)


# speedup vs baseline: 2.4912x; 1.1683x over previous
import functools

import jax
import jax.numpy as jnp
from jax import lax
from jax.experimental import pallas as pl
from jax.experimental.pallas import tpu as pltpu
from jax.experimental.pallas import tpu_sc as plsc

N_NODES = 10000
N_EDGES = 320000
D = 128

NC = 2
NS = 16
NW = NC * NS
EPW = N_EDGES // NW
K = 80
G = 25
NG = EPW // (G * K)
NCH = NG * G
NP = 10240
RPS = NP // NS
ZR = 80

K3 = 80
G3 = 25
NG3 = EPW // (G3 * K3)

_mesh = plsc.VectorSubcoreMesh(
    core_axis_name="c", subcore_axis_name="s", num_cores=NC, num_subcores=NS
)


NBUF = 4


def _sc_aggregate_body(x_hbm, eidx_hbm, out_hbm,
                       sidx_v, didx_v, rows0, rows1, rows2, rows3,
                       acc, sem0, sem1, sem2, sem3):
    c = lax.axis_index("c")
    s = lax.axis_index("s")
    wid = s * NC + c
    rows = (rows0, rows1, rows2, rows3)
    sems = (sem0, sem1, sem2, sem3)

    z16 = jnp.zeros((16,), jnp.float32)

    @pl.loop(0, ZR)
    def _fill(i):
        for j in range(D // 16):
            rows0[i, pl.ds(j * 16, 16)] = z16

    for t in range(RPS // ZR):
        pltpu.sync_copy(rows0, acc.at[pl.ds(s * RPS + t * ZR, ZR)])
    plsc.subcore_barrier()

    @pl.loop(0, NG)
    def _group(g):
        pltpu.sync_copy(eidx_hbm.at[0, wid, g], sidx_v)
        pltpu.sync_copy(eidx_hbm.at[1, wid, g], didx_v)
        for b in range(NBUF):
            pltpu.async_copy(x_hbm.at[sidx_v.at[b]], rows[b], sems[b])

        @pl.loop(0, (G - NBUF - 1) // NBUF)
        def _quad(i):
            j0 = NBUF * i
            for b in range(NBUF):
                pltpu.make_async_copy(x_hbm.at[sidx_v.at[j0 + b]], rows[b],
                                      sems[b]).wait()
                pltpu.sync_copy(rows[b], acc.at[didx_v.at[j0 + b]], add=True)
                pltpu.async_copy(x_hbm.at[sidx_v.at[j0 + NBUF + b]], rows[b],
                                 sems[b])

        pltpu.make_async_copy(x_hbm.at[sidx_v.at[G - 5]], rows[0],
                              sems[0]).wait()
        pltpu.sync_copy(rows[0], acc.at[didx_v.at[G - 5]], add=True)
        pltpu.async_copy(x_hbm.at[sidx_v.at[G - 1]], rows[0], sems[0])
        for b in range(1, NBUF):
            pltpu.make_async_copy(x_hbm.at[sidx_v.at[G - 5 + b]], rows[b],
                                  sems[b]).wait()
            pltpu.sync_copy(rows[b], acc.at[didx_v.at[G - 5 + b]], add=True)
        pltpu.make_async_copy(x_hbm.at[sidx_v.at[G - 1]], rows[0],
                              sems[0]).wait()
        pltpu.sync_copy(rows[0], acc.at[didx_v.at[G - 1]], add=True)

    plsc.subcore_barrier()
    pltpu.sync_copy(acc.at[pl.ds(s * RPS, RPS)],
                    out_hbm.at[c, pl.ds(s * RPS, RPS)])


_sc_aggregate = pl.kernel(
    _sc_aggregate_body,
    out_type=jax.ShapeDtypeStruct((NC, NP, D), jnp.float32),
    mesh=_mesh,
    scratch_types=[
        pltpu.VMEM((G, K), jnp.int32),
        pltpu.VMEM((G, K), jnp.int32),
        pltpu.VMEM((K, D), jnp.float32),
        pltpu.VMEM((K, D), jnp.float32),
        pltpu.VMEM((K, D), jnp.float32),
        pltpu.VMEM((K, D), jnp.float32),
        pltpu.MemorySpace.VMEM_SHARED((NP, D), jnp.float32),
        pltpu.SemaphoreType.DMA,
        pltpu.SemaphoreType.DMA,
        pltpu.SemaphoreType.DMA,
        pltpu.SemaphoreType.DMA,
    ],
)


def _tc_mlp_body(scale_ref, x_ref, p0_ref, p1_ref, w1_ref, b1_ref,
                 w2_ref, b2_ref, wfcT_ref, bsT_ref, s_ref):
    h = x_ref[...] * scale_ref[0] + p0_ref[0] + p1_ref[0]
    h = jnp.maximum(
        jnp.dot(h, w1_ref[...], preferred_element_type=jnp.float32)
        + b1_ref[...], 0.0)
    h = jnp.maximum(
        jnp.dot(h, w2_ref[...], preferred_element_type=jnp.float32)
        + b2_ref[...], 0.0)
    s_ref[...] = (
        lax.dot_general(wfcT_ref[...], h, (((1,), (1,)), ((), ())),
                        preferred_element_type=jnp.float32)
        + bsT_ref[...])


_RB = 1024

_tc_mlp = pl.pallas_call(
    _tc_mlp_body,
    grid=(pl.cdiv(N_NODES, _RB),),
    in_specs=[
        pl.BlockSpec(memory_space=pltpu.MemorySpace.SMEM),
        pl.BlockSpec((_RB, D), lambda i: (i, 0)),
        pl.BlockSpec((1, _RB, D), lambda i: (0, i, 0)),
        pl.BlockSpec((1, _RB, D), lambda i: (1, i, 0)),
        pl.BlockSpec((D, D), lambda i: (0, 0)),
        pl.BlockSpec((1, D), lambda i: (0, 0)),
        pl.BlockSpec((D, D), lambda i: (0, 0)),
        pl.BlockSpec((1, D), lambda i: (0, 0)),
        pl.BlockSpec((8, D), lambda i: (0, 0)),
        pl.BlockSpec((8, 1), lambda i: (0, 0)),
    ],
    out_specs=pl.BlockSpec((8, _RB), lambda i: (0, i)),
    out_shape=jax.ShapeDtypeStruct((8, N_NODES), jnp.float32),
)


def _sc_scores_body(st_hbm, eidx_hbm, out_hbm,
                    sidx_v, didx_v, out_v, s1_sh, s2_sh, sem0, sem1):
    c = lax.axis_index("c")
    s = lax.axis_index("s")
    wid = s * NC + c

    @pl.when(s == 0)
    def _stage():
        pltpu.sync_copy(st_hbm.at[0], s1_sh)
        pltpu.sync_copy(st_hbm.at[1], s2_sh)

    plsc.subcore_barrier()

    @pl.loop(0, NG3)
    def _group(g):
        pltpu.sync_copy(eidx_hbm.at[0, wid, g], sidx_v)
        pltpu.sync_copy(eidx_hbm.at[1, wid, g], didx_v)
        base = g * G3 * K3

        def _sl(j):
            return out_v.at[pl.ds(base + j * K3, K3)]

        pltpu.async_copy(s1_sh.at[sidx_v.at[0]], _sl(0), sem0)
        pltpu.async_copy(s1_sh.at[sidx_v.at[1]], _sl(1), sem1)

        @pl.loop(0, (G3 - 3) // 2)
        def _pair(i):
            j0 = 2 * i
            pltpu.make_async_copy(s1_sh.at[sidx_v.at[j0]], _sl(j0),
                                  sem0).wait()
            pltpu.async_copy(s1_sh.at[sidx_v.at[j0 + 2]], _sl(j0 + 2), sem0)
            pltpu.sync_copy(s2_sh.at[didx_v.at[j0]], _sl(j0), add=True)
            pltpu.make_async_copy(s1_sh.at[sidx_v.at[j0 + 1]], _sl(j0 + 1),
                                  sem1).wait()
            pltpu.async_copy(s1_sh.at[sidx_v.at[j0 + 3]], _sl(j0 + 3), sem1)
            pltpu.sync_copy(s2_sh.at[didx_v.at[j0 + 1]], _sl(j0 + 1),
                            add=True)

        pltpu.make_async_copy(s1_sh.at[sidx_v.at[G3 - 3]], _sl(G3 - 3),
                              sem0).wait()
        pltpu.async_copy(s1_sh.at[sidx_v.at[G3 - 1]], _sl(G3 - 1), sem0)
        pltpu.sync_copy(s2_sh.at[didx_v.at[G3 - 3]], _sl(G3 - 3), add=True)
        pltpu.make_async_copy(s1_sh.at[sidx_v.at[G3 - 2]], _sl(G3 - 2),
                              sem1).wait()
        pltpu.sync_copy(s2_sh.at[didx_v.at[G3 - 2]], _sl(G3 - 2), add=True)
        pltpu.make_async_copy(s1_sh.at[sidx_v.at[G3 - 1]], _sl(G3 - 1),
                              sem0).wait()
        pltpu.sync_copy(s2_sh.at[didx_v.at[G3 - 1]], _sl(G3 - 1), add=True)

    pltpu.sync_copy(out_v, out_hbm.at[pl.ds(wid * EPW, EPW)])


_sc_scores = pl.kernel(
    _sc_scores_body,
    out_type=jax.ShapeDtypeStruct((N_EDGES,), jnp.float32),
    mesh=_mesh,
    scratch_types=[
        pltpu.VMEM((G3, K3), jnp.int32),
        pltpu.VMEM((G3, K3), jnp.int32),
        pltpu.VMEM((EPW,), jnp.float32),
        pltpu.MemorySpace.VMEM_SHARED((N_NODES,), jnp.float32),
        pltpu.MemorySpace.VMEM_SHARED((N_NODES,), jnp.float32),
        pltpu.SemaphoreType.DMA,
        pltpu.SemaphoreType.DMA,
    ],
)


def kernel(x, edge_index, eps, W1, b1, W2, b2, Wfc, bfc):
    edge_r = edge_index.astype(jnp.int32).reshape(2, NW, NG, G, K)

    parts = _sc_aggregate(x, edge_r)

    scale = (1.0 + eps).reshape(1).astype(jnp.float32)
    wfcT = jnp.pad(Wfc.reshape(2, D), ((0, 6), (0, 0)))
    bsT = jnp.zeros((8, 1), jnp.float32).at[0, 0].set(bfc[0])
    S_T = _tc_mlp(scale, x, parts, parts, W1, b1.reshape(1, D),
                  W2, b2.reshape(1, D), wfcT, bsT)

    return _sc_scores(S_T, edge_r)


# speedup vs baseline: 2.6712x; 1.0722x over previous
import functools

import jax
import jax.numpy as jnp
from jax import lax
from jax.experimental import pallas as pl
from jax.experimental.pallas import tpu as pltpu
from jax.experimental.pallas import tpu_sc as plsc

N_NODES = 10000
N_EDGES = 320000
D = 128

NC = 2
NS = 16
NW = NC * NS
EPW = N_EDGES // NW
K = 80
G = 25
NG = EPW // (G * K)
NCH = NG * G
NP = 10240
RPS = NP // NS
ZR = 80

K3 = 80
G3 = 25
NG3 = EPW // (G3 * K3)

_mesh = plsc.VectorSubcoreMesh(
    core_axis_name="c", subcore_axis_name="s", num_cores=NC, num_subcores=NS
)


NBUF = 4


def _sc_aggregate_body(x_hbm, eidx_hbm, out_hbm,
                       sidx_v, didx_v, rows0, rows1, rows2, rows3,
                       acc, sem0, sem1, sem2, sem3):
    c = lax.axis_index("c")
    s = lax.axis_index("s")
    wid = s * NC + c
    rows = (rows0, rows1, rows2, rows3)
    sems = (sem0, sem1, sem2, sem3)

    z16 = jnp.zeros((16,), jnp.float32)

    @pl.loop(0, ZR)
    def _fill(i):
        for j in range(D // 16):
            rows0[i, pl.ds(j * 16, 16)] = z16

    for t in range(RPS // ZR):
        pltpu.sync_copy(rows0, acc.at[pl.ds(s * RPS + t * ZR, ZR)])
    plsc.subcore_barrier()

    @pl.loop(0, NG)
    def _group(g):
        pltpu.sync_copy(eidx_hbm.at[0, wid, g], sidx_v)
        pltpu.sync_copy(eidx_hbm.at[1, wid, g], didx_v)
        for b in range(NBUF):
            pltpu.async_copy(x_hbm.at[sidx_v.at[b]], rows[b], sems[b])

        @pl.loop(0, (G - NBUF - 1) // NBUF)
        def _quad(i):
            j0 = NBUF * i
            for b in range(NBUF):
                pltpu.make_async_copy(x_hbm.at[sidx_v.at[j0 + b]], rows[b],
                                      sems[b]).wait()
                pltpu.sync_copy(rows[b], acc.at[didx_v.at[j0 + b]], add=True)
                pltpu.async_copy(x_hbm.at[sidx_v.at[j0 + NBUF + b]], rows[b],
                                 sems[b])

        pltpu.make_async_copy(x_hbm.at[sidx_v.at[G - 5]], rows[0],
                              sems[0]).wait()
        pltpu.sync_copy(rows[0], acc.at[didx_v.at[G - 5]], add=True)
        pltpu.async_copy(x_hbm.at[sidx_v.at[G - 1]], rows[0], sems[0])
        for b in range(1, NBUF):
            pltpu.make_async_copy(x_hbm.at[sidx_v.at[G - 5 + b]], rows[b],
                                  sems[b]).wait()
            pltpu.sync_copy(rows[b], acc.at[didx_v.at[G - 5 + b]], add=True)
        pltpu.make_async_copy(x_hbm.at[sidx_v.at[G - 1]], rows[0],
                              sems[0]).wait()
        pltpu.sync_copy(rows[0], acc.at[didx_v.at[G - 1]], add=True)

    plsc.subcore_barrier()
    pltpu.sync_copy(acc.at[pl.ds(s * RPS, RPS)],
                    out_hbm.at[c, pl.ds(s * RPS, RPS)])


_sc_aggregate = pl.kernel(
    _sc_aggregate_body,
    out_type=jax.ShapeDtypeStruct((NC, NP, D), jnp.float32),
    mesh=_mesh,
    scratch_types=[
        pltpu.VMEM((G, K), jnp.int32),
        pltpu.VMEM((G, K), jnp.int32),
        pltpu.VMEM((K, D), jnp.float32),
        pltpu.VMEM((K, D), jnp.float32),
        pltpu.VMEM((K, D), jnp.float32),
        pltpu.VMEM((K, D), jnp.float32),
        pltpu.MemorySpace.VMEM_SHARED((NP, D), jnp.float32),
        pltpu.SemaphoreType.DMA,
        pltpu.SemaphoreType.DMA,
        pltpu.SemaphoreType.DMA,
        pltpu.SemaphoreType.DMA,
    ],
)


def _tc_mlp_body(scale_ref, x_ref, p0_ref, p1_ref, w1_ref, b1_ref,
                 w2_ref, b2_ref, wfcT_ref, bsT_ref, s_ref):
    h = x_ref[...] * scale_ref[0] + p0_ref[0] + p1_ref[0]
    h = jnp.maximum(
        jnp.dot(h, w1_ref[...], preferred_element_type=jnp.float32)
        + b1_ref[...], 0.0)
    h = jnp.maximum(
        jnp.dot(h, w2_ref[...], preferred_element_type=jnp.float32)
        + b2_ref[...], 0.0)
    s_ref[...] = (
        lax.dot_general(wfcT_ref[...], h, (((1,), (1,)), ((), ())),
                        preferred_element_type=jnp.float32)
        + bsT_ref[...])


_RB = 1024

_tc_mlp = pl.pallas_call(
    _tc_mlp_body,
    grid=(pl.cdiv(N_NODES, _RB),),
    in_specs=[
        pl.BlockSpec(memory_space=pltpu.MemorySpace.SMEM),
        pl.BlockSpec((_RB, D), lambda i: (i, 0)),
        pl.BlockSpec((1, _RB, D), lambda i: (0, i, 0)),
        pl.BlockSpec((1, _RB, D), lambda i: (1, i, 0)),
        pl.BlockSpec((D, D), lambda i: (0, 0)),
        pl.BlockSpec((1, D), lambda i: (0, 0)),
        pl.BlockSpec((D, D), lambda i: (0, 0)),
        pl.BlockSpec((1, D), lambda i: (0, 0)),
        pl.BlockSpec((8, D), lambda i: (0, 0)),
        pl.BlockSpec((8, 1), lambda i: (0, 0)),
    ],
    out_specs=pl.BlockSpec((8, _RB), lambda i: (0, i)),
    out_shape=jax.ShapeDtypeStruct((8, N_NODES), jnp.float32),
)


def _sc_scores_body(st_hbm, eidx_hbm, out_hbm,
                    sidx_v, didx_v, out_v, s1_sh, s2_sh,
                    sem0, sem1, sem2, sem3, sem4, sem5):
    c = lax.axis_index("c")
    s = lax.axis_index("s")
    wid = s * NC + c

    @pl.when(s == 0)
    def _stage():
        pltpu.sync_copy(st_hbm.at[0], s1_sh)
        pltpu.sync_copy(st_hbm.at[1], s2_sh)

    plsc.subcore_barrier()

    @pl.loop(0, NG3)
    def _group(g):
        pltpu.sync_copy(eidx_hbm.at[0, wid, g], sidx_v)
        pltpu.sync_copy(eidx_hbm.at[1, wid, g], didx_v)
        base = g * G3 * K3
        s1sems = (sem0, sem1, sem2)
        s2sems = (sem3, sem4, sem5)

        def _sl(j):
            return out_v.at[pl.ds(base + j * K3, K3)]

        for j in range(3):
            pltpu.async_copy(s1_sh.at[sidx_v.at[j]], _sl(j), s1sems[j])

        @pl.loop(0, (G3 - 4) // 3)
        def _trip(i):
            j0 = 3 * i
            for b in range(3):
                j = j0 + b
                pltpu.make_async_copy(s1_sh.at[sidx_v.at[j]], _sl(j),
                                      s1sems[b]).wait()
                @pl.when(i > 0)
                def _():
                    pltpu.make_async_copy(s2_sh.at[didx_v.at[j - 3]],
                                          _sl(j - 3), s2sems[b]).wait()
                pltpu.async_copy(s2_sh.at[didx_v.at[j]], _sl(j), s2sems[b],
                                 add=True)
                pltpu.async_copy(s1_sh.at[sidx_v.at[j + 3]], _sl(j + 3),
                                 s1sems[b])

        for j in range(G3 - 4, G3):
            b = j % 3
            if j == G3 - 1:
                pltpu.async_copy(s1_sh.at[sidx_v.at[j]], _sl(j), s1sems[b])
            pltpu.make_async_copy(s1_sh.at[sidx_v.at[j]], _sl(j),
                                  s1sems[b]).wait()
            pltpu.make_async_copy(s2_sh.at[didx_v.at[j - 3]], _sl(j - 3),
                                  s2sems[b]).wait()
            pltpu.async_copy(s2_sh.at[didx_v.at[j]], _sl(j), s2sems[b],
                             add=True)
        for j in range(G3 - 3, G3):
            b = j % 3
            pltpu.make_async_copy(s2_sh.at[didx_v.at[j]], _sl(j),
                                  s2sems[b]).wait()

    pltpu.sync_copy(out_v, out_hbm.at[pl.ds(wid * EPW, EPW)])


_sc_scores = pl.kernel(
    _sc_scores_body,
    out_type=jax.ShapeDtypeStruct((N_EDGES,), jnp.float32),
    mesh=_mesh,
    scratch_types=[
        pltpu.VMEM((G3, K3), jnp.int32),
        pltpu.VMEM((G3, K3), jnp.int32),
        pltpu.VMEM((EPW,), jnp.float32),
        pltpu.MemorySpace.VMEM_SHARED((N_NODES,), jnp.float32),
        pltpu.MemorySpace.VMEM_SHARED((N_NODES,), jnp.float32),
        pltpu.SemaphoreType.DMA,
        pltpu.SemaphoreType.DMA,
        pltpu.SemaphoreType.DMA,
        pltpu.SemaphoreType.DMA,
        pltpu.SemaphoreType.DMA,
        pltpu.SemaphoreType.DMA,
    ],
)


def kernel(x, edge_index, eps, W1, b1, W2, b2, Wfc, bfc):
    edge_r = edge_index.astype(jnp.int32).reshape(2, NW, NG, G, K)

    parts = _sc_aggregate(x, edge_r)

    scale = (1.0 + eps).reshape(1).astype(jnp.float32)
    wfcT = jnp.pad(Wfc.reshape(2, D), ((0, 6), (0, 0)))
    bsT = jnp.zeros((8, 1), jnp.float32).at[0, 0].set(bfc[0])
    S_T = _tc_mlp(scale, x, parts, parts, W1, b1.reshape(1, D),
                  W2, b2.reshape(1, D), wfcT, bsT)

    return _sc_scores(S_T, edge_r)


# speedup vs baseline: 2.7186x; 1.0177x over previous
import functools

import jax
import jax.numpy as jnp
from jax import lax
from jax.experimental import pallas as pl
from jax.experimental.pallas import tpu as pltpu
from jax.experimental.pallas import tpu_sc as plsc

N_NODES = 10000
N_EDGES = 320000
D = 128

NC = 2
NS = 16
NW = NC * NS
EPW = N_EDGES // NW
K = 80
G = 25
NG = EPW // (G * K)
NCH = NG * G
NP = 10240
RPS = NP // NS
ZR = 80

K3 = 80
G3 = 25
NG3 = EPW // (G3 * K3)

_mesh = plsc.VectorSubcoreMesh(
    core_axis_name="c", subcore_axis_name="s", num_cores=NC, num_subcores=NS
)


NBUF = 4


def _sc_aggregate_body(x_hbm, eidx_hbm, out_hbm,
                       sidx_v, didx_v, rows0, rows1, rows2, rows3,
                       acc, sem0, sem1, sem2, sem3):
    c = lax.axis_index("c")
    s = lax.axis_index("s")
    wid = s * NC + c
    rows = (rows0, rows1, rows2, rows3)
    sems = (sem0, sem1, sem2, sem3)

    z16 = jnp.zeros((16,), jnp.float32)

    @pl.loop(0, ZR)
    def _fill(i):
        for j in range(D // 16):
            rows0[i, pl.ds(j * 16, 16)] = z16

    for t in range(RPS // ZR):
        pltpu.sync_copy(rows0, acc.at[pl.ds(s * RPS + t * ZR, ZR)])
    plsc.subcore_barrier()

    @pl.loop(0, NG)
    def _group(g):
        pltpu.sync_copy(eidx_hbm.at[0, wid, g], sidx_v)
        pltpu.sync_copy(eidx_hbm.at[1, wid, g], didx_v)
        for b in range(NBUF):
            pltpu.async_copy(x_hbm.at[sidx_v.at[b]], rows[b], sems[b])

        @pl.loop(0, (G - NBUF - 1) // NBUF)
        def _quad(i):
            j0 = NBUF * i
            for b in range(NBUF):
                pltpu.make_async_copy(x_hbm.at[sidx_v.at[j0 + b]], rows[b],
                                      sems[b]).wait()
                pltpu.sync_copy(rows[b], acc.at[didx_v.at[j0 + b]], add=True)
                pltpu.async_copy(x_hbm.at[sidx_v.at[j0 + NBUF + b]], rows[b],
                                 sems[b])

        pltpu.make_async_copy(x_hbm.at[sidx_v.at[G - 5]], rows[0],
                              sems[0]).wait()
        pltpu.sync_copy(rows[0], acc.at[didx_v.at[G - 5]], add=True)
        pltpu.async_copy(x_hbm.at[sidx_v.at[G - 1]], rows[0], sems[0])
        for b in range(1, NBUF):
            pltpu.make_async_copy(x_hbm.at[sidx_v.at[G - 5 + b]], rows[b],
                                  sems[b]).wait()
            pltpu.sync_copy(rows[b], acc.at[didx_v.at[G - 5 + b]], add=True)
        pltpu.make_async_copy(x_hbm.at[sidx_v.at[G - 1]], rows[0],
                              sems[0]).wait()
        pltpu.sync_copy(rows[0], acc.at[didx_v.at[G - 1]], add=True)

    plsc.subcore_barrier()
    pltpu.sync_copy(acc.at[pl.ds(s * RPS, RPS)],
                    out_hbm.at[c, pl.ds(s * RPS, RPS)])


_sc_aggregate = pl.kernel(
    _sc_aggregate_body,
    out_type=jax.ShapeDtypeStruct((NC, NP, D), jnp.float32),
    mesh=_mesh,
    scratch_types=[
        pltpu.VMEM((G, K), jnp.int32),
        pltpu.VMEM((G, K), jnp.int32),
        pltpu.VMEM((K, D), jnp.float32),
        pltpu.VMEM((K, D), jnp.float32),
        pltpu.VMEM((K, D), jnp.float32),
        pltpu.VMEM((K, D), jnp.float32),
        pltpu.MemorySpace.VMEM_SHARED((NP, D), jnp.float32),
        pltpu.SemaphoreType.DMA,
        pltpu.SemaphoreType.DMA,
        pltpu.SemaphoreType.DMA,
        pltpu.SemaphoreType.DMA,
    ],
)


def _tc_mlp_body(scale_ref, x_ref, p0_ref, p1_ref, w1_ref, b1_ref,
                 w2_ref, b2_ref, wfcT_ref, bsT_ref, s_ref):
    h = x_ref[...] * scale_ref[0] + p0_ref[0] + p1_ref[0]
    h = jnp.maximum(
        jnp.dot(h, w1_ref[...], preferred_element_type=jnp.float32)
        + b1_ref[...], 0.0)
    h = jnp.maximum(
        jnp.dot(h, w2_ref[...], preferred_element_type=jnp.float32)
        + b2_ref[...], 0.0)
    s_ref[...] = (
        lax.dot_general(wfcT_ref[...], h, (((1,), (1,)), ((), ())),
                        preferred_element_type=jnp.float32)
        + bsT_ref[...])


_RB = 2048

_tc_mlp = pl.pallas_call(
    _tc_mlp_body,
    grid=(pl.cdiv(N_NODES, _RB),),
    in_specs=[
        pl.BlockSpec(memory_space=pltpu.MemorySpace.SMEM),
        pl.BlockSpec((_RB, D), lambda i: (i, 0)),
        pl.BlockSpec((1, _RB, D), lambda i: (0, i, 0)),
        pl.BlockSpec((1, _RB, D), lambda i: (1, i, 0)),
        pl.BlockSpec((D, D), lambda i: (0, 0)),
        pl.BlockSpec((1, D), lambda i: (0, 0)),
        pl.BlockSpec((D, D), lambda i: (0, 0)),
        pl.BlockSpec((1, D), lambda i: (0, 0)),
        pl.BlockSpec((8, D), lambda i: (0, 0)),
        pl.BlockSpec((8, 1), lambda i: (0, 0)),
    ],
    out_specs=pl.BlockSpec((8, _RB), lambda i: (0, i)),
    out_shape=jax.ShapeDtypeStruct((8, N_NODES), jnp.float32),
)


def _sc_scores_body(st_hbm, eidx_hbm, out_hbm,
                    sidx_v, didx_v, out_v, s1_sh, s2_sh,
                    sem0, sem1, sem2, sem3, sem4, sem5):
    c = lax.axis_index("c")
    s = lax.axis_index("s")
    wid = s * NC + c

    @pl.when(s == 0)
    def _stage():
        pltpu.sync_copy(st_hbm.at[0], s1_sh)
        pltpu.sync_copy(st_hbm.at[1], s2_sh)

    plsc.subcore_barrier()

    @pl.loop(0, NG3)
    def _group(g):
        pltpu.sync_copy(eidx_hbm.at[0, wid, g], sidx_v)
        pltpu.sync_copy(eidx_hbm.at[1, wid, g], didx_v)
        base = g * G3 * K3
        s1sems = (sem0, sem1, sem2)
        s2sems = (sem3, sem4, sem5)

        def _sl(j):
            return out_v.at[pl.ds(base + j * K3, K3)]

        for j in range(3):
            pltpu.async_copy(s1_sh.at[sidx_v.at[j]], _sl(j), s1sems[j])

        @pl.loop(0, (G3 - 4) // 3)
        def _trip(i):
            j0 = 3 * i
            for b in range(3):
                j = j0 + b
                pltpu.make_async_copy(s1_sh.at[sidx_v.at[j]], _sl(j),
                                      s1sems[b]).wait()
                @pl.when(i > 0)
                def _():
                    pltpu.make_async_copy(s2_sh.at[didx_v.at[j - 3]],
                                          _sl(j - 3), s2sems[b]).wait()
                pltpu.async_copy(s2_sh.at[didx_v.at[j]], _sl(j), s2sems[b],
                                 add=True)
                pltpu.async_copy(s1_sh.at[sidx_v.at[j + 3]], _sl(j + 3),
                                 s1sems[b])

        for j in range(G3 - 4, G3):
            b = j % 3
            if j == G3 - 1:
                pltpu.async_copy(s1_sh.at[sidx_v.at[j]], _sl(j), s1sems[b])
            pltpu.make_async_copy(s1_sh.at[sidx_v.at[j]], _sl(j),
                                  s1sems[b]).wait()
            pltpu.make_async_copy(s2_sh.at[didx_v.at[j - 3]], _sl(j - 3),
                                  s2sems[b]).wait()
            pltpu.async_copy(s2_sh.at[didx_v.at[j]], _sl(j), s2sems[b],
                             add=True)
        for j in range(G3 - 3, G3):
            b = j % 3
            pltpu.make_async_copy(s2_sh.at[didx_v.at[j]], _sl(j),
                                  s2sems[b]).wait()

    pltpu.sync_copy(out_v, out_hbm.at[pl.ds(wid * EPW, EPW)])


_sc_scores = pl.kernel(
    _sc_scores_body,
    out_type=jax.ShapeDtypeStruct((N_EDGES,), jnp.float32),
    mesh=_mesh,
    scratch_types=[
        pltpu.VMEM((G3, K3), jnp.int32),
        pltpu.VMEM((G3, K3), jnp.int32),
        pltpu.VMEM((EPW,), jnp.float32),
        pltpu.MemorySpace.VMEM_SHARED((N_NODES,), jnp.float32),
        pltpu.MemorySpace.VMEM_SHARED((N_NODES,), jnp.float32),
        pltpu.SemaphoreType.DMA,
        pltpu.SemaphoreType.DMA,
        pltpu.SemaphoreType.DMA,
        pltpu.SemaphoreType.DMA,
        pltpu.SemaphoreType.DMA,
        pltpu.SemaphoreType.DMA,
    ],
)


def kernel(x, edge_index, eps, W1, b1, W2, b2, Wfc, bfc):
    edge_r = edge_index.astype(jnp.int32).reshape(2, NW, NG, G, K)

    parts = _sc_aggregate(x, edge_r)

    scale = (1.0 + eps).reshape(1).astype(jnp.float32)
    wfcT = jnp.pad(Wfc.reshape(2, D), ((0, 6), (0, 0)))
    bsT = jnp.zeros((8, 1), jnp.float32).at[0, 0].set(bfc[0])
    S_T = _tc_mlp(scale, x, parts, parts, W1, b1.reshape(1, D),
                  W2, b2.reshape(1, D), wfcT, bsT)

    return _sc_scores(S_T, edge_r)
